# baseline (speedup 1.0000x reference)
import functools

import jax
import jax.numpy as jnp
from jax import lax
from jax.experimental import pallas as pl
from jax.experimental.pallas import tpu as pltpu
from jax.experimental.pallas import tpu_sc as plsc

N = 10000
D = 256
E = 160000

NC = 2
NS = 16
CH = 128
E_PAD = 163840
CHUNKS = E_PAD // CH
CPT = CHUNKS // NS
NG = 2
GC = CPT // NG
RA = 10240
RPT = RA // NS
XROWS = 2 * RA
BLK = 400
NBLK = N // BLK

_mesh = plsc.VectorSubcoreMesh(core_axis_name="c", subcore_axis_name="s")


@functools.partial(
    pl.kernel,
    out_type=jax.ShapeDtypeStruct((2, RA, 128), jnp.float32),
    mesh=_mesh,
    scratch_types=[
        pltpu.VMEM((CPT, CH), jnp.int32),
        pltpu.VMEM((CH, 128), jnp.float32),
        pltpu.VMEM_SHARED((RA, 128), jnp.float32),
        pltpu.SemaphoreType.DMA,
    ],
)
def _deg_kernel(e_hbm, ones_hbm, zer_hbm, out_hbm, idx_v, ones_v, acc, sem):
    c = lax.axis_index("c")
    s = lax.axis_index("s")
    pltpu.sync_copy(ones_hbm, ones_v)
    pltpu.sync_copy(zer_hbm, acc.at[pl.ds(s * RPT, RPT)])
    pltpu.sync_copy(e_hbm.at[c, pl.ds(s * CPT, CPT)], idx_v)
    plsc.subcore_barrier()

    def body(j, carry):
        pltpu.sync_copy(ones_v, acc.at[idx_v.at[j]], add=True)
        return carry

    lax.fori_loop(0, CPT, body, 0)
    plsc.subcore_barrier()
    pltpu.sync_copy(acc.at[pl.ds(s * RPT, RPT)],
                    out_hbm.at[c, pl.ds(s * RPT, RPT)])


@functools.partial(
    pl.kernel,
    out_type=jax.ShapeDtypeStruct((2, RA, 128), jnp.float32),
    mesh=_mesh,
    scratch_types=[
        pltpu.VMEM((GC, CH), jnp.int32),
        pltpu.VMEM((GC, CH), jnp.int32),
        pltpu.VMEM((CH, 128), jnp.float32),
        pltpu.VMEM((CH, 128), jnp.float32),
        pltpu.VMEM_SHARED((RA, 128), jnp.float32),
        pltpu.SemaphoreType.DMA,
        pltpu.SemaphoreType.DMA,
    ],
)
def _agg_kernel(sidx_hbm, didx_hbm, xs_hbm, zer_hbm, out_hbm,
                sv, dv, ra, rb, acc, sem_a, sem_b):
    c = lax.axis_index("c")
    s = lax.axis_index("s")
    pltpu.sync_copy(zer_hbm, acc.at[pl.ds(s * RPT, RPT)])
    plsc.subcore_barrier()
    for g in range(NG):
        base = s * CPT + g * GC
        pltpu.sync_copy(sidx_hbm.at[c, pl.ds(base, GC)], sv)
        pltpu.sync_copy(didx_hbm.at[pl.ds(base, GC)], dv)
        pltpu.async_copy(xs_hbm.at[sv.at[0]], ra, sem_a)

        def body(p, carry):
            j = 2 * p
            pltpu.async_copy(xs_hbm.at[sv.at[j + 1]], rb, sem_b)
            pltpu.make_async_copy(xs_hbm.at[sv.at[0]], ra, sem_a).wait()
            pltpu.sync_copy(ra, acc.at[dv.at[j]], add=True)

            @pl.when(p + 1 < GC // 2)
            def _():
                pltpu.async_copy(xs_hbm.at[sv.at[j + 2]], ra, sem_a)

            pltpu.make_async_copy(xs_hbm.at[sv.at[0]], rb, sem_b).wait()
            pltpu.sync_copy(rb, acc.at[dv.at[j + 1]], add=True)
            return carry

        lax.fori_loop(0, GC // 2, body, 0)
    plsc.subcore_barrier()
    pltpu.sync_copy(acc.at[pl.ds(s * RPT, RPT)],
                    out_hbm.at[c, pl.ds(s * RPT, RPT)])


@functools.partial(
    pl.kernel,
    out_type=jax.ShapeDtypeStruct((2, 8, 128), jnp.float32),
    mesh=_mesh,
    scratch_types=[
        pltpu.VMEM((40, 64), jnp.int32),
        [pltpu.VMEM((64, 128), jnp.float32)] * 5,
        [pltpu.SemaphoreType.DMA] * 5,
    ],
)
def _probe_g3(sidx_hbm, xs_hbm, out_hbm, sv, bufs, gsem):
    c = lax.axis_index("c")
    s = lax.axis_index("s")

    def gwait(b):
        pltpu.make_async_copy(xs_hbm.at[sv.at[0]], bufs[b], gsem[b]).wait()

    for g in range(4):
        base = s * 160 + g * 40
        pltpu.sync_copy(sidx_hbm.at[c, pl.ds(base, 40)], sv)
        for b in range(3):
            pltpu.async_copy(xs_hbm.at[sv.at[b]], bufs[b], gsem[b])

        def body(q, carry):
            for k in range(5):
                i = 5 * q + k
                gwait(k)

                @pl.when(i + 3 < 40)
                def _():
                    pltpu.async_copy(xs_hbm.at[sv.at[i + 3]],
                                     bufs[(k + 3) % 5], gsem[(k + 3) % 5])
            return carry

        lax.fori_loop(0, 8, body, 0)

    @pl.when(s == 0)
    def _():
        pltpu.sync_copy(bufs[0].at[pl.ds(0, 8)], out_hbm.at[c])


def _prep_body(x_ref, dego_ref, xs_ref):
    ns = lax.rsqrt(jnp.maximum(dego_ref[...], 1.0))
    xs_ref[...] = (x_ref[...] * ns).reshape(2 * BLK, 128)


def _l1_body(a0_ref, a1_ref, degi_ref, dego_ref, w_ref, b_ref, h_ref,
             hs_ref):
    nd = lax.rsqrt(jnp.maximum(degi_ref[...], 1.0))
    a0 = a0_ref[0] * nd
    a1 = a1_ref[0] * nd
    acc = jnp.dot(a0, w_ref[0:128, :], preferred_element_type=jnp.float32)
    acc = acc + jnp.dot(a1, w_ref[128:256, :],
                        preferred_element_type=jnp.float32)
    h = jnp.maximum(acc + b_ref[...], 0.0)
    h_ref[...] = h
    ns = lax.rsqrt(jnp.maximum(dego_ref[...], 1.0))
    hs_ref[...] = (h * ns).reshape(2 * BLK, 128)


def _l2_body(a0_ref, a1_ref, degi_ref, w_ref, b_ref, h_ref):
    nd = lax.rsqrt(jnp.maximum(degi_ref[...], 1.0))
    a0 = a0_ref[0] * nd
    a1 = a1_ref[0] * nd
    acc = jnp.dot(a0, w_ref[0:128, :], preferred_element_type=jnp.float32)
    acc = acc + jnp.dot(a1, w_ref[128:256, :],
                        preferred_element_type=jnp.float32)
    h_ref[...] = acc + b_ref[...]


_D1_SPEC = pl.BlockSpec((BLK, 1), lambda i: (i, 0))
_FLAT_SPEC = pl.BlockSpec((2 * BLK, 128), lambda i: (i, 0))
_A0_SPEC = pl.BlockSpec((1, BLK, 128), lambda i: (0, i, 0))
_A1_SPEC = pl.BlockSpec((1, BLK, 128), lambda i: (1, i, 0))
_ROW_SPEC = pl.BlockSpec((BLK, D), lambda i: (i, 0))
_W_SPEC = pl.BlockSpec((D, D), lambda i: (0, 0))
_B_SPEC = pl.BlockSpec((1, D), lambda i: (0, 0))


def _prep(x, deg_out):
    return pl.pallas_call(
        _prep_body,
        grid=(NBLK,),
        in_specs=[_ROW_SPEC, _D1_SPEC],
        out_specs=_FLAT_SPEC,
        out_shape=jax.ShapeDtypeStruct((XROWS, 128), jnp.float32),
    )(x, deg_out)


def _layer1(agg, deg_in, deg_out, w, b):
    return pl.pallas_call(
        _l1_body,
        grid=(NBLK,),
        in_specs=[_A0_SPEC, _A1_SPEC, _D1_SPEC, _D1_SPEC, _W_SPEC, _B_SPEC],
        out_specs=[_ROW_SPEC, _FLAT_SPEC],
        out_shape=[
            jax.ShapeDtypeStruct((N, D), jnp.float32),
            jax.ShapeDtypeStruct((XROWS, 128), jnp.float32),
        ],
    )(agg, agg, deg_in, deg_out, w, b)


def _layer2(agg, deg_in, w, b):
    return pl.pallas_call(
        _l2_body,
        grid=(NBLK,),
        in_specs=[_A0_SPEC, _A1_SPEC, _D1_SPEC, _W_SPEC, _B_SPEC],
        out_specs=_ROW_SPEC,
        out_shape=jax.ShapeDtypeStruct((N, D), jnp.float32),
    )(agg, agg, deg_in, w, b)


def kernel(x, edge_index, W1, b1, W2, b2):
    src = edge_index[0].astype(jnp.int32)
    dst = edge_index[1].astype(jnp.int32)
    pad = E_PAD - E
    fill = jnp.full((pad,), N, jnp.int32)
    src_p = jnp.concatenate([src, fill])
    dst_p = jnp.concatenate([dst, fill])
    e2d = jnp.stack([src_p, dst_p]).reshape(2, CHUNKS, CH)
    srcg = jnp.stack([2 * src_p, 2 * src_p + 1]).reshape(2, CHUNKS, CH)
    dstg = dst_p.reshape(CHUNKS, CH)
    ones_c = jnp.ones((CH, 128), jnp.float32)
    zer_acc = jnp.zeros((RPT, 128), jnp.float32)
    b1r = b1.reshape(1, D)
    b2r = b2.reshape(1, D)

    deg = _deg_kernel(e2d, ones_c, zer_acc)
    deg_out = deg[0, :, 0:1]
    deg_in = deg[1, :, 0:1]
    xs = _prep(x, deg_out)
    _go = _probe_g3(srcg.reshape(2, 2560, 64), xs)
    agg1 = _agg_kernel(srcg, dstg, xs + 0.0 * _go[0, 0, 0], zer_acc)
    h1, h1s = _layer1(agg1, deg_in, deg_out, W1, b1r)
    agg2 = _agg_kernel(srcg, dstg, h1s, zer_acc)
    h2 = _layer2(agg2, deg_in, W2, b2r)
    return (x, h1, h2)


# speedup vs baseline: 1.6084x; 1.6084x over previous
import functools

import jax
import jax.numpy as jnp
from jax import lax
from jax.experimental import pallas as pl
from jax.experimental.pallas import tpu as pltpu
from jax.experimental.pallas import tpu_sc as plsc

N = 10000
D = 256
E = 160000

NC = 2
NS = 16
CH = 128
E_PAD = 163840
CHUNKS = E_PAD // CH
CPT = CHUNKS // NS
NG = 2
GC = CPT // NG
RA = 10240
RPT = RA // NS
XROWS = 2 * RA
BLK = 400
NBLK = N // BLK

_mesh = plsc.VectorSubcoreMesh(core_axis_name="c", subcore_axis_name="s")


@functools.partial(
    pl.kernel,
    out_type=jax.ShapeDtypeStruct((2, RA, 128), jnp.float32),
    mesh=_mesh,
    scratch_types=[
        pltpu.VMEM((CPT, CH), jnp.int32),
        pltpu.VMEM((CH, 128), jnp.float32),
        pltpu.VMEM_SHARED((RA, 128), jnp.float32),
        pltpu.SemaphoreType.DMA,
    ],
)
def _deg_kernel(e_hbm, ones_hbm, zer_hbm, out_hbm, idx_v, ones_v, acc, sem):
    c = lax.axis_index("c")
    s = lax.axis_index("s")
    pltpu.sync_copy(ones_hbm, ones_v)
    pltpu.sync_copy(zer_hbm, acc.at[pl.ds(s * RPT, RPT)])
    pltpu.sync_copy(e_hbm.at[c, pl.ds(s * CPT, CPT)], idx_v)
    plsc.subcore_barrier()

    def body(j, carry):
        pltpu.sync_copy(ones_v, acc.at[idx_v.at[j]], add=True)
        return carry

    lax.fori_loop(0, CPT, body, 0)
    plsc.subcore_barrier()
    pltpu.sync_copy(acc.at[pl.ds(s * RPT, RPT)],
                    out_hbm.at[c, pl.ds(s * RPT, RPT)])


@functools.partial(
    pl.kernel,
    out_type=jax.ShapeDtypeStruct((2, RA, 128), jnp.float32),
    mesh=_mesh,
    scratch_types=[
        pltpu.VMEM((GC, CH), jnp.int32),
        pltpu.VMEM((GC, CH), jnp.int32),
        pltpu.VMEM((CH, 128), jnp.float32),
        pltpu.VMEM((CH, 128), jnp.float32),
        pltpu.VMEM_SHARED((RA, 128), jnp.float32),
        pltpu.SemaphoreType.DMA,
        pltpu.SemaphoreType.DMA,
    ],
)
def _agg_kernel(sidx_hbm, didx_hbm, xs_hbm, zer_hbm, out_hbm,
                sv, dv, ra, rb, acc, sem_a, sem_b):
    c = lax.axis_index("c")
    s = lax.axis_index("s")
    pltpu.sync_copy(zer_hbm, acc.at[pl.ds(s * RPT, RPT)])
    plsc.subcore_barrier()
    for g in range(NG):
        base = s * CPT + g * GC
        pltpu.sync_copy(sidx_hbm.at[c, pl.ds(base, GC)], sv)
        pltpu.sync_copy(didx_hbm.at[pl.ds(base, GC)], dv)
        pltpu.async_copy(xs_hbm.at[sv.at[0]], ra, sem_a)

        def body(p, carry):
            j = 2 * p
            pltpu.async_copy(xs_hbm.at[sv.at[j + 1]], rb, sem_b)
            pltpu.make_async_copy(xs_hbm.at[sv.at[0]], ra, sem_a).wait()
            pltpu.sync_copy(ra, acc.at[dv.at[j]], add=True)

            @pl.when(p + 1 < GC // 2)
            def _():
                pltpu.async_copy(xs_hbm.at[sv.at[j + 2]], ra, sem_a)

            pltpu.make_async_copy(xs_hbm.at[sv.at[0]], rb, sem_b).wait()
            pltpu.sync_copy(rb, acc.at[dv.at[j + 1]], add=True)
            return carry

        lax.fori_loop(0, GC // 2, body, 0)
    plsc.subcore_barrier()
    pltpu.sync_copy(acc.at[pl.ds(s * RPT, RPT)],
                    out_hbm.at[c, pl.ds(s * RPT, RPT)])


def _prep_body(x_ref, dego_ref, xs_ref):
    ns = lax.rsqrt(jnp.maximum(dego_ref[...], 1.0))
    xs_ref[...] = (x_ref[...] * ns).reshape(2 * BLK, 128)


def _l1_body(a0_ref, a1_ref, degi_ref, dego_ref, w_ref, b_ref, h_ref,
             hs_ref):
    nd = lax.rsqrt(jnp.maximum(degi_ref[...], 1.0))
    a0 = a0_ref[0] * nd
    a1 = a1_ref[0] * nd
    acc = jnp.dot(a0, w_ref[0:128, :], preferred_element_type=jnp.float32)
    acc = acc + jnp.dot(a1, w_ref[128:256, :],
                        preferred_element_type=jnp.float32)
    h = jnp.maximum(acc + b_ref[...], 0.0)
    h_ref[...] = h
    ns = lax.rsqrt(jnp.maximum(dego_ref[...], 1.0))
    hs_ref[...] = (h * ns).reshape(2 * BLK, 128)


def _l2_body(a0_ref, a1_ref, degi_ref, w_ref, b_ref, h_ref):
    nd = lax.rsqrt(jnp.maximum(degi_ref[...], 1.0))
    a0 = a0_ref[0] * nd
    a1 = a1_ref[0] * nd
    acc = jnp.dot(a0, w_ref[0:128, :], preferred_element_type=jnp.float32)
    acc = acc + jnp.dot(a1, w_ref[128:256, :],
                        preferred_element_type=jnp.float32)
    h_ref[...] = acc + b_ref[...]


_D1_SPEC = pl.BlockSpec((BLK, 1), lambda i: (i, 0))
_FLAT_SPEC = pl.BlockSpec((2 * BLK, 128), lambda i: (i, 0))
_A0_SPEC = pl.BlockSpec((1, BLK, 128), lambda i: (0, i, 0))
_A1_SPEC = pl.BlockSpec((1, BLK, 128), lambda i: (1, i, 0))
_ROW_SPEC = pl.BlockSpec((BLK, D), lambda i: (i, 0))
_W_SPEC = pl.BlockSpec((D, D), lambda i: (0, 0))
_B_SPEC = pl.BlockSpec((1, D), lambda i: (0, 0))


def _prep(x, deg_out):
    return pl.pallas_call(
        _prep_body,
        grid=(NBLK,),
        in_specs=[_ROW_SPEC, _D1_SPEC],
        out_specs=_FLAT_SPEC,
        out_shape=jax.ShapeDtypeStruct((XROWS, 128), jnp.float32),
    )(x, deg_out)


def _layer1(agg, deg_in, deg_out, w, b):
    return pl.pallas_call(
        _l1_body,
        grid=(NBLK,),
        in_specs=[_A0_SPEC, _A1_SPEC, _D1_SPEC, _D1_SPEC, _W_SPEC, _B_SPEC],
        out_specs=[_ROW_SPEC, _FLAT_SPEC],
        out_shape=[
            jax.ShapeDtypeStruct((N, D), jnp.float32),
            jax.ShapeDtypeStruct((XROWS, 128), jnp.float32),
        ],
    )(agg, agg, deg_in, deg_out, w, b)


def _layer2(agg, deg_in, w, b):
    return pl.pallas_call(
        _l2_body,
        grid=(NBLK,),
        in_specs=[_A0_SPEC, _A1_SPEC, _D1_SPEC, _W_SPEC, _B_SPEC],
        out_specs=_ROW_SPEC,
        out_shape=jax.ShapeDtypeStruct((N, D), jnp.float32),
    )(agg, agg, deg_in, w, b)


def kernel(x, edge_index, W1, b1, W2, b2):
    src = edge_index[0].astype(jnp.int32)
    dst = edge_index[1].astype(jnp.int32)
    pad = E_PAD - E
    fill = jnp.full((pad,), N, jnp.int32)
    src_p = jnp.concatenate([src, fill])
    dst_p = jnp.concatenate([dst, fill])
    e2d = jnp.stack([src_p, dst_p]).reshape(2, CHUNKS, CH)
    srcg = jnp.stack([2 * src_p, 2 * src_p + 1]).reshape(2, CHUNKS, CH)
    dstg = dst_p.reshape(CHUNKS, CH)
    ones_c = jnp.ones((CH, 128), jnp.float32)
    zer_acc = jnp.zeros((RPT, 128), jnp.float32)
    b1r = b1.reshape(1, D)
    b2r = b2.reshape(1, D)

    deg = _deg_kernel(e2d, ones_c, zer_acc)
    deg_out = deg[0, :, 0:1]
    deg_in = deg[1, :, 0:1]
    xs = _prep(x, deg_out)
    agg1 = _agg_kernel(srcg, dstg, xs, zer_acc)
    h1, h1s = _layer1(agg1, deg_in, deg_out, W1, b1r)
    agg2 = _agg_kernel(srcg, dstg, h1s, zer_acc)
    h2 = _layer2(agg2, deg_in, W2, b2r)
    return (x, h1, h2)
